# baseline (speedup 1.0000x reference)
import functools

import jax
import jax.numpy as jnp
import numpy as np
from jax.experimental import pallas as pl
from jax.experimental.pallas import tpu as pltpu

_B, _NB, _D, _N = 8, 4, 256, 1024
_CD, _BINS = 512, 1024
_BPAIR = 8

_HI = jax.lax.Precision.HIGHEST


def _vq_body(x_ref, pit_ref, cb_ref, pot_ref,
             quant_ref, codes_ref, losssum_ref,
             cbphl_ref, cbsq_ref):
    q = pl.program_id(0)
    bb = pl.program_id(1)

    @pl.when(bb == 0)
    def _():
        cb = cb_ref[0]
        pot = pot_ref[0]
        cbp = jax.lax.dot_general(
            pot, cb, (((1,), (1,)), ((), ())), precision=_HI)
        hi = cbp.astype(jnp.bfloat16).astype(jnp.float32)
        cbphl_ref[0:_D] = hi
        cbphl_ref[_D:2 * _D] = cbp - hi
        cbsq_ref[...] = jnp.sum(cb * cb, axis=1, keepdims=True)

    @pl.when(jnp.logical_and(q == 0, bb == 0))
    def _():
        losssum_ref[...] = jnp.zeros_like(losssum_ref)

    pit = pit_ref[0]
    iotaf = jax.lax.broadcasted_iota(
        jnp.int32, (_BINS, _N), 0).astype(jnp.float32)
    dn = (((1,), (0,)), ((), ()))
    loss = None
    for j in range(_BPAIR):
        x = x_ref[j, 0]
        z_t = jax.lax.dot_general(pit, x, dn)
        z_sq = jnp.sum(z_t * z_t, axis=0, keepdims=True)
        dots_t = jax.lax.dot_general(cb_ref[0], z_t, dn)
        dist_t = z_sq - 2.0 * dots_t + cbsq_ref[...]

        mind = jnp.min(dist_t, axis=0, keepdims=True)
        codesf = jnp.min(jnp.where(dist_t == mind, iotaf, float(_BINS)),
                         axis=0, keepdims=True)

        onehot = (iotaf == codesf).astype(jnp.float32)
        r = jax.lax.dot_general(cbphl_ref[...], onehot, dn)
        quant_ref[j, 0] = r[0:_D] + r[_D:2 * _D]
        codes_ref[j, 0] = codesf.astype(jnp.int32)
        s = jnp.sum(mind)
        loss = s if loss is None else loss + s

    losssum_ref[...] += loss


@jax.jit
def _vq_call(x, pit, cb, pot):
    grid = (_NB, _B // _BPAIR)
    quant, codes4, losssum = pl.pallas_call(
        _vq_body,
        grid=grid,
        in_specs=[
            pl.BlockSpec((_BPAIR, 1, _D, _N), lambda q, bb: (bb, q, 0, 0)),
            pl.BlockSpec((1, _CD, _D), lambda q, bb: (q, 0, 0)),
            pl.BlockSpec((1, _BINS, _CD), lambda q, bb: (q, 0, 0)),
            pl.BlockSpec((1, _D, _CD), lambda q, bb: (q, 0, 0)),
        ],
        out_specs=[
            pl.BlockSpec((_BPAIR, 1, _D, _N), lambda q, bb: (bb, q, 0, 0)),
            pl.BlockSpec((_BPAIR, 1, 1, _N), lambda q, bb: (bb, q, 0, 0)),
            pl.BlockSpec((1, 1), lambda q, bb: (0, 0)),
        ],
        out_shape=[
            jax.ShapeDtypeStruct((_B, _NB, _D, _N), jnp.float32),
            jax.ShapeDtypeStruct((_B, _NB, 1, _N), jnp.int32),
            jax.ShapeDtypeStruct((1, 1), jnp.float32),
        ],
        scratch_shapes=[
            pltpu.VMEM((2 * _D, _BINS), jnp.float32),
            pltpu.VMEM((_BINS, 1), jnp.float32),
        ],
        compiler_params=pltpu.CompilerParams(
            dimension_semantics=("arbitrary", "arbitrary"),
        ),
    )(x, pit, cb, pot)
    return quant, codes4, losssum


def kernel(x, sample_rate, proj_in, proj_out, codebook):
    pit = jnp.transpose(proj_in, (0, 2, 1))
    pot = jnp.transpose(proj_out, (0, 2, 1))
    quant, codes4, losssum = _vq_call(x, pit, codebook, pot)
    codes_out = codes4.reshape(_B, _NB, _N)
    loss = losssum[0, 0] / np.float32(_NB * _B * _N * _CD)
    bw = jnp.asarray(_NB * (np.log2(_BINS) * sample_rate / 1000.0),
                     dtype=x.dtype)
    return quant, codes_out, bw, loss


# speedup vs baseline: 1.0130x; 1.0130x over previous
import functools

import jax
import jax.numpy as jnp
import numpy as np
from jax.experimental import pallas as pl
from jax.experimental.pallas import tpu as pltpu

_B, _NB, _D, _N = 8, 4, 256, 1024
_CD, _BINS = 512, 1024
_BPAIR = 4

_HI = jax.lax.Precision.HIGHEST


def _vq_body(x_ref, pit_ref, cb_ref, pot_ref,
             quant_ref, codes_ref, losssum_ref,
             cbphl_ref, cbsq_ref):
    q = pl.program_id(0)
    bb = pl.program_id(1)

    @pl.when(bb == 0)
    def _():
        cb = cb_ref[0]
        pot = pot_ref[0]
        cbp = jax.lax.dot_general(
            pot, cb, (((1,), (1,)), ((), ())), precision=_HI)
        hi = cbp.astype(jnp.bfloat16).astype(jnp.float32)
        cbphl_ref[0:_D] = hi
        cbphl_ref[_D:2 * _D] = cbp - hi
        cbsq_ref[...] = jnp.sum(cb * cb, axis=1, keepdims=True)

    @pl.when(jnp.logical_and(q == 0, bb == 0))
    def _():
        losssum_ref[...] = jnp.zeros_like(losssum_ref)

    pit = pit_ref[0]
    iotaf = jax.lax.broadcasted_iota(
        jnp.int32, (_BINS, _N), 0).astype(jnp.float32)
    dn = (((1,), (0,)), ((), ()))
    loss = None
    for j in range(_BPAIR):
        x = x_ref[j, 0]
        z_t = jax.lax.dot_general(pit, x, dn)
        z_sq = jnp.sum(z_t * z_t, axis=0, keepdims=True)
        dots_t = jax.lax.dot_general(cb_ref[0], z_t, dn)
        dist_t = z_sq - 2.0 * dots_t + cbsq_ref[...]

        mind = jnp.min(dist_t, axis=0, keepdims=True)
        codesf = jnp.min(jnp.where(dist_t == mind, iotaf, float(_BINS)),
                         axis=0, keepdims=True)

        onehot = (iotaf == codesf).astype(jnp.float32)
        r = jax.lax.dot_general(cbphl_ref[...], onehot, dn)
        quant_ref[j, 0] = r[0:_D] + r[_D:2 * _D]
        codes_ref[j, 0] = codesf.astype(jnp.int32)
        s = jnp.sum(mind)
        loss = s if loss is None else loss + s

    losssum_ref[...] += loss


@jax.jit
def _vq_call(x, pit, cb, pot):
    grid = (_NB, _B // _BPAIR)
    quant, codes4, losssum = pl.pallas_call(
        _vq_body,
        grid=grid,
        in_specs=[
            pl.BlockSpec((_BPAIR, 1, _D, _N), lambda q, bb: (bb, q, 0, 0)),
            pl.BlockSpec((1, _CD, _D), lambda q, bb: (q, 0, 0)),
            pl.BlockSpec((1, _BINS, _CD), lambda q, bb: (q, 0, 0)),
            pl.BlockSpec((1, _D, _CD), lambda q, bb: (q, 0, 0)),
        ],
        out_specs=[
            pl.BlockSpec((_BPAIR, 1, _D, _N), lambda q, bb: (bb, q, 0, 0)),
            pl.BlockSpec((_BPAIR, 1, 1, _N), lambda q, bb: (bb, q, 0, 0)),
            pl.BlockSpec((1, 1), lambda q, bb: (0, 0)),
        ],
        out_shape=[
            jax.ShapeDtypeStruct((_B, _NB, _D, _N), jnp.float32),
            jax.ShapeDtypeStruct((_B, _NB, 1, _N), jnp.int32),
            jax.ShapeDtypeStruct((1, 1), jnp.float32),
        ],
        scratch_shapes=[
            pltpu.VMEM((2 * _D, _BINS), jnp.float32),
            pltpu.VMEM((_BINS, 1), jnp.float32),
        ],
        compiler_params=pltpu.CompilerParams(
            dimension_semantics=("arbitrary", "arbitrary"),
        ),
    )(x, pit, cb, pot)
    return quant, codes4, losssum


def kernel(x, sample_rate, proj_in, proj_out, codebook):
    pit = jnp.transpose(proj_in, (0, 2, 1))
    pot = jnp.transpose(proj_out, (0, 2, 1))
    quant, codes4, losssum = _vq_call(x, pit, codebook, pot)
    codes_out = codes4.reshape(_B, _NB, _N)
    loss = losssum[0, 0] / np.float32(_NB * _B * _N * _CD)
    bw = jnp.asarray(_NB * (np.log2(_BINS) * sample_rate / 1000.0),
                     dtype=x.dtype)
    return quant, codes_out, bw, loss
